# baseline (speedup 1.0000x reference)
import functools

import jax
import jax.numpy as jnp
from jax import lax
from jax.experimental import pallas as pl
from jax.experimental.pallas import tpu as pltpu
from jax.experimental.pallas import tpu_sc as plsc

D_IN = 1024
D_OUT = 1024
NUM_EXPERTS = 8
T = 2048
A = 2 * T
R = A + T

TILE = 256
TILE_M = 128
N_TILES = A // TILE_M
MAX_STEPS = N_TILES + NUM_EXPERTS - 1

NC = 2
NS = 16
NW = NC * NS
TW = T // NW


def _gating_kernel(x_ref, gate_w_ref, weight_w_ref,
                   p0_ref, p1_ref, c0_ref, c1_ref, cr_ref, offs_ref):
    x = x_ref[...]
    logits = jnp.dot(x, gate_w_ref[...], preferred_element_type=jnp.float32)
    probs = jax.nn.softmax(logits, axis=-1)
    lane = jax.lax.broadcasted_iota(jnp.int32, probs.shape, 1)
    i1 = jnp.argmax(probs, axis=-1)[:, None]
    v1 = jnp.max(probs, axis=-1, keepdims=True)
    probs2 = jnp.where(lane == i1, -1.0, probs)
    i2 = jnp.argmax(probs2, axis=-1)[:, None]
    v2 = jnp.max(probs2, axis=-1, keepdims=True)
    oh1 = (lane == i1).astype(jnp.float32)
    oh2 = (lane == i2).astype(jnp.float32)

    cnt = oh1 + oh2
    ri = jax.lax.broadcasted_iota(jnp.int32, (T, T), 0)
    ci = jax.lax.broadcasted_iota(jnp.int32, (T, T), 1)
    tri = (ci <= ri).astype(jnp.bfloat16)
    cs = jnp.dot(tri, cnt.astype(jnp.bfloat16),
                 preferred_element_type=jnp.float32)
    excl = cs - cnt
    totals = cs[T - 1:T, :]

    base1 = jnp.sum(jnp.where(lane < i1, totals, 0.0), axis=-1, keepdims=True)
    base2 = jnp.sum(jnp.where(lane < i2, totals, 0.0), axis=-1, keepdims=True)
    rank1 = jnp.sum(oh1 * excl, axis=-1, keepdims=True)
    rank2 = jnp.sum(oh2 * excl, axis=-1, keepdims=True)
    p0 = base1 + rank1
    p1 = base2 + rank2
    p0_ref[...] = p0.astype(jnp.int32)
    p1_ref[...] = p1.astype(jnp.int32)

    wlog = jnp.dot(x, weight_w_ref[...], preferred_element_type=jnp.float32)
    ow = jax.nn.softmax(wlog, axis=-1)
    ow0 = ow[:, 0:1]
    c0_ref[...] = ow0 * v1
    c1_ref[...] = ow0 * v2
    cr_ref[...] = ow[:, 1:2]

    offs_ref[...] = jnp.concatenate([totals, totals * 0.0],
                                    axis=1).astype(jnp.int32)


def _gating(x, gate_w, weight_w):
    return pl.pallas_call(
        _gating_kernel,
        grid=(1,),
        in_specs=[
            pl.BlockSpec((T, D_IN), lambda i: (0, 0)),
            pl.BlockSpec((D_IN, NUM_EXPERTS), lambda i: (0, 0)),
            pl.BlockSpec((D_IN, 2), lambda i: (0, 0)),
        ],
        out_specs=[
            pl.BlockSpec((T, 1), lambda i: (0, 0)),
            pl.BlockSpec((T, 1), lambda i: (0, 0)),
            pl.BlockSpec((T, 1), lambda i: (0, 0)),
            pl.BlockSpec((T, 1), lambda i: (0, 0)),
            pl.BlockSpec((T, 1), lambda i: (0, 0)),
            pl.BlockSpec((1, 16), lambda i: (0, 0)),
        ],
        out_shape=[
            jax.ShapeDtypeStruct((T, 1), jnp.int32),
            jax.ShapeDtypeStruct((T, 1), jnp.int32),
            jax.ShapeDtypeStruct((T, 1), jnp.float32),
            jax.ShapeDtypeStruct((T, 1), jnp.float32),
            jax.ShapeDtypeStruct((T, 1), jnp.float32),
            jax.ShapeDtypeStruct((1, 16), jnp.int32),
        ],
    )(x, gate_w, weight_w)


def _scatter_body(x_hbm, p0_hbm, p1_hbm, gx_hbm, idx0_v, idx1_v, rows_v, sem):
    wid = lax.axis_index("s") * NC + lax.axis_index("c")
    base = wid * TW
    pltpu.sync_copy(p0_hbm.at[pl.ds(base, TW)], idx0_v)
    pltpu.sync_copy(p1_hbm.at[pl.ds(base, TW)], idx1_v)
    pltpu.sync_copy(x_hbm.at[pl.ds(base, TW)], rows_v)
    pltpu.async_copy(rows_v, gx_hbm.at[idx0_v], sem).wait()
    pltpu.async_copy(rows_v, gx_hbm.at[idx1_v], sem).wait()


def _sc_scatter(x, p0, p1):
    mesh = plsc.VectorSubcoreMesh(core_axis_name="c", subcore_axis_name="s")
    return pl.kernel(
        _scatter_body,
        out_type=jax.ShapeDtypeStruct((A, D_IN), jnp.float32),
        mesh=mesh,
        scratch_types=[
            pltpu.VMEM((TW,), jnp.int32),
            pltpu.VMEM((TW,), jnp.int32),
            pltpu.VMEM((TW, D_IN), jnp.float32),
            pltpu.SemaphoreType.DMA,
        ],
    )(x, p0, p1)


def _gmm_kernel(tile_r, grp_r, valid_r, offs_r,
                gx_ref, ew_ref, eb_ref, y_ref):
    i = pl.program_id(0)
    t = tile_r[i]
    g = grp_r[i]
    first = jnp.logical_or(i == 0, t != tile_r[jnp.maximum(i - 1, 0)])

    @pl.when(first)
    def _zero():
        y_ref[...] = jnp.zeros_like(y_ref)

    seg0 = offs_r[g]
    seg1 = offs_r[g + 1]
    rows = t * TILE_M + jax.lax.broadcasted_iota(jnp.int32, (TILE_M, 1), 0)
    msk = ((rows >= seg0) & (rows < seg1)).astype(jnp.float32)

    @pl.when(valid_r[i] == 1)
    def _expert():
        sub = jax.lax.broadcasted_iota(jnp.int32, (NUM_EXPERTS, 1), 0)
        eb = jnp.sum(jnp.where(sub == g, eb_ref[...], 0.0), axis=0,
                     keepdims=True)
        y = jnp.dot(gx_ref[...], ew_ref[0],
                    preferred_element_type=jnp.float32) + eb
        y_ref[...] += msk * y


def _moe_matmul(tile_id, grp_id, valid, offs, gx, expert_w, expert_b):
    grid_spec = pltpu.PrefetchScalarGridSpec(
        num_scalar_prefetch=4,
        grid=(MAX_STEPS,),
        in_specs=[
            pl.BlockSpec((TILE_M, D_IN),
                         lambda i, tr, gr, vr, orf: (tr[i], 0)),
            pl.BlockSpec((1, D_IN, D_OUT),
                         lambda i, tr, gr, vr, orf: (gr[i], 0, 0)),
            pl.BlockSpec((NUM_EXPERTS, D_OUT),
                         lambda i, tr, gr, vr, orf: (0, 0)),
        ],
        out_specs=pl.BlockSpec((TILE_M, D_OUT),
                               lambda i, tr, gr, vr, orf: (tr[i], 0)),
    )
    return pl.pallas_call(
        _gmm_kernel,
        grid_spec=grid_spec,
        out_shape=jax.ShapeDtypeStruct((A, D_OUT), jnp.float32),
        compiler_params=pltpu.CompilerParams(
            dimension_semantics=("arbitrary",)),
    )(tile_id, grp_id, valid, offs, gx, expert_w, expert_b)


def _res_kernel(x_ref, rw_ref, rb_ref, cr_ref, yres_ref):
    y = jnp.dot(x_ref[...], rw_ref[...],
                preferred_element_type=jnp.float32) + rb_ref[...][None, :]
    yres_ref[...] = cr_ref[...] * y


def _res_matmul(x, res_w, res_b, cr):
    m_tiles = T // TILE
    return pl.pallas_call(
        _res_kernel,
        grid=(m_tiles,),
        in_specs=[
            pl.BlockSpec((TILE, D_IN), lambda m: (m, 0)),
            pl.BlockSpec((D_IN, D_OUT), lambda m: (0, 0)),
            pl.BlockSpec((D_OUT,), lambda m: (0,)),
            pl.BlockSpec((TILE, 1), lambda m: (m, 0)),
        ],
        out_specs=pl.BlockSpec((TILE, D_OUT), lambda m: (m, 0)),
        out_shape=jax.ShapeDtypeStruct((T, D_OUT), jnp.float32),
    )(x, res_w, res_b, cr)


def _gather_body(y_hbm, p0_hbm, p1_hbm, y1_hbm, y2_hbm,
                 idx0_v, idx1_v, rows_v, sem):
    wid = lax.axis_index("s") * NC + lax.axis_index("c")
    base = wid * TW
    pltpu.sync_copy(p0_hbm.at[pl.ds(base, TW)], idx0_v)
    pltpu.sync_copy(p1_hbm.at[pl.ds(base, TW)], idx1_v)
    pltpu.async_copy(y_hbm.at[idx0_v], rows_v, sem).wait()
    pltpu.sync_copy(rows_v, y1_hbm.at[pl.ds(base, TW)])
    pltpu.async_copy(y_hbm.at[idx1_v], rows_v, sem).wait()
    pltpu.sync_copy(rows_v, y2_hbm.at[pl.ds(base, TW)])


def _sc_gather(y, p0, p1):
    mesh = plsc.VectorSubcoreMesh(core_axis_name="c", subcore_axis_name="s")
    return pl.kernel(
        _gather_body,
        out_type=[
            jax.ShapeDtypeStruct((T, D_OUT), jnp.float32),
            jax.ShapeDtypeStruct((T, D_OUT), jnp.float32),
        ],
        mesh=mesh,
        scratch_types=[
            pltpu.VMEM((TW,), jnp.int32),
            pltpu.VMEM((TW,), jnp.int32),
            pltpu.VMEM((TW, D_OUT), jnp.float32),
            pltpu.SemaphoreType.DMA,
        ],
    )(y, p0, p1)


def _combine_kernel(y1_ref, y2_ref, yres_ref, c0_ref, c1_ref, out_ref):
    out_ref[...] = (c0_ref[...] * y1_ref[...] + c1_ref[...] * y2_ref[...]
                    + yres_ref[...])


def _combine(y1, y2, yres, c0, c1):
    m_tiles = T // TILE
    return pl.pallas_call(
        _combine_kernel,
        grid=(m_tiles,),
        in_specs=[
            pl.BlockSpec((TILE, D_OUT), lambda m: (m, 0)),
            pl.BlockSpec((TILE, D_OUT), lambda m: (m, 0)),
            pl.BlockSpec((TILE, D_OUT), lambda m: (m, 0)),
            pl.BlockSpec((TILE, 1), lambda m: (m, 0)),
            pl.BlockSpec((TILE, 1), lambda m: (m, 0)),
        ],
        out_specs=pl.BlockSpec((TILE, D_OUT), lambda m: (m, 0)),
        out_shape=jax.ShapeDtypeStruct((T, D_OUT), jnp.float32),
    )(y1, y2, yres, c0, c1)


def _step_list(offs_v):
    t_ar = jnp.arange(N_TILES, dtype=jnp.int32)
    g_ar = jnp.arange(NUM_EXPERTS, dtype=jnp.int32)
    ts = t_ar[:, None] * TILE_M
    s0 = offs_v[None, :NUM_EXPERTS]
    s1 = offs_v[None, 1:NUM_EXPERTS + 1]
    big = jnp.int32(2 ** 30)
    m = (s0 < ts + TILE_M) & (s1 > ts)
    keys = jnp.where(m, t_ar[:, None] * 16 + g_ar[None, :], big).reshape(-1)
    keys = jnp.sort(keys)[:MAX_STEPS]
    valid = (keys < big).astype(jnp.int32)
    tile_id = jnp.where(valid == 1, keys >> 4, N_TILES - 1).astype(jnp.int32)
    grp_id = jnp.where(valid == 1, keys & 15, NUM_EXPERTS - 1).astype(jnp.int32)
    return tile_id, grp_id, valid


def kernel(x, gate_w, expert_w, expert_b, res_gate_w, res_w, res_b, weight_w):
    del res_gate_w
    p0_2d, p1_2d, c0_2d, c1_2d, cr_2d, counts16 = _gating(x, gate_w, weight_w)
    p0 = p0_2d.reshape(T)
    p1 = p1_2d.reshape(T)
    counts = counts16[0, :NUM_EXPERTS]
    offs_v = jnp.concatenate([
        jnp.zeros((1,), jnp.int32),
        jnp.cumsum(counts),
    ])
    tile_id, grp_id, valid = _step_list(offs_v)
    gx = _sc_scatter(x, p0, p1)
    yres = _res_matmul(x, res_w, res_b, cr_2d)
    y = _moe_matmul(tile_id, grp_id, valid, offs_v, gx, expert_w, expert_b)
    y1, y2 = _sc_gather(y, p0, p1)
    return _combine(y1, y2, yres, c0_2d, c1_2d)


# speedup vs baseline: 1.0741x; 1.0741x over previous
import functools

import jax
import jax.numpy as jnp
from jax import lax
from jax.experimental import pallas as pl
from jax.experimental.pallas import tpu as pltpu
from jax.experimental.pallas import tpu_sc as plsc

D_IN = 1024
D_OUT = 1024
NUM_EXPERTS = 8
T = 2048
A = 2 * T
R = A + T

TILE = 256
TILE_M = 256
N_TILES = A // TILE_M
MAX_STEPS = N_TILES + NUM_EXPERTS - 1

NC = 2
NS = 16
NW = NC * NS
TW = T // NW


def _gating_kernel(x_ref, gate_w_ref, weight_w_ref,
                   p0_ref, p1_ref, c0_ref, c1_ref, cr_ref, offs_ref):
    x = x_ref[...]
    logits = jnp.dot(x, gate_w_ref[...], preferred_element_type=jnp.float32)
    probs = jax.nn.softmax(logits, axis=-1)
    lane = jax.lax.broadcasted_iota(jnp.int32, probs.shape, 1)
    i1 = jnp.argmax(probs, axis=-1)[:, None]
    v1 = jnp.max(probs, axis=-1, keepdims=True)
    probs2 = jnp.where(lane == i1, -1.0, probs)
    i2 = jnp.argmax(probs2, axis=-1)[:, None]
    v2 = jnp.max(probs2, axis=-1, keepdims=True)
    oh1 = (lane == i1).astype(jnp.float32)
    oh2 = (lane == i2).astype(jnp.float32)

    cnt = oh1 + oh2
    ri = jax.lax.broadcasted_iota(jnp.int32, (T, T), 0)
    ci = jax.lax.broadcasted_iota(jnp.int32, (T, T), 1)
    tri = (ci <= ri).astype(jnp.bfloat16)
    cs = jnp.dot(tri, cnt.astype(jnp.bfloat16),
                 preferred_element_type=jnp.float32)
    excl = cs - cnt
    totals = cs[T - 1:T, :]

    base1 = jnp.sum(jnp.where(lane < i1, totals, 0.0), axis=-1, keepdims=True)
    base2 = jnp.sum(jnp.where(lane < i2, totals, 0.0), axis=-1, keepdims=True)
    rank1 = jnp.sum(oh1 * excl, axis=-1, keepdims=True)
    rank2 = jnp.sum(oh2 * excl, axis=-1, keepdims=True)
    p0 = base1 + rank1
    p1 = base2 + rank2
    p0_ref[...] = p0.astype(jnp.int32)
    p1_ref[...] = p1.astype(jnp.int32)

    wlog = jnp.dot(x, weight_w_ref[...], preferred_element_type=jnp.float32)
    ow = jax.nn.softmax(wlog, axis=-1)
    ow0 = ow[:, 0:1]
    c0_ref[...] = ow0 * v1
    c1_ref[...] = ow0 * v2
    cr_ref[...] = ow[:, 1:2]

    offs_ref[...] = jnp.concatenate([totals, totals * 0.0],
                                    axis=1).astype(jnp.int32)


def _gating(x, gate_w, weight_w):
    return pl.pallas_call(
        _gating_kernel,
        grid=(1,),
        in_specs=[
            pl.BlockSpec((T, D_IN), lambda i: (0, 0)),
            pl.BlockSpec((D_IN, NUM_EXPERTS), lambda i: (0, 0)),
            pl.BlockSpec((D_IN, 2), lambda i: (0, 0)),
        ],
        out_specs=[
            pl.BlockSpec((T, 1), lambda i: (0, 0)),
            pl.BlockSpec((T, 1), lambda i: (0, 0)),
            pl.BlockSpec((T, 1), lambda i: (0, 0)),
            pl.BlockSpec((T, 1), lambda i: (0, 0)),
            pl.BlockSpec((T, 1), lambda i: (0, 0)),
            pl.BlockSpec((1, 16), lambda i: (0, 0)),
        ],
        out_shape=[
            jax.ShapeDtypeStruct((T, 1), jnp.int32),
            jax.ShapeDtypeStruct((T, 1), jnp.int32),
            jax.ShapeDtypeStruct((T, 1), jnp.float32),
            jax.ShapeDtypeStruct((T, 1), jnp.float32),
            jax.ShapeDtypeStruct((T, 1), jnp.float32),
            jax.ShapeDtypeStruct((1, 16), jnp.int32),
        ],
    )(x, gate_w, weight_w)


def _scatter_body(x_hbm, p0_hbm, p1_hbm, gx_hbm, idx0_v, idx1_v, rows_v, sem):
    wid = lax.axis_index("s") * NC + lax.axis_index("c")
    base = wid * TW
    pltpu.sync_copy(p0_hbm.at[pl.ds(base, TW)], idx0_v)
    pltpu.sync_copy(p1_hbm.at[pl.ds(base, TW)], idx1_v)
    pltpu.sync_copy(x_hbm.at[pl.ds(base, TW)], rows_v)
    pltpu.async_copy(rows_v, gx_hbm.at[idx0_v], sem).wait()
    pltpu.async_copy(rows_v, gx_hbm.at[idx1_v], sem).wait()


def _sc_scatter(x, p0, p1):
    mesh = plsc.VectorSubcoreMesh(core_axis_name="c", subcore_axis_name="s")
    return pl.kernel(
        _scatter_body,
        out_type=jax.ShapeDtypeStruct((A, D_IN), jnp.float32),
        mesh=mesh,
        scratch_types=[
            pltpu.VMEM((TW,), jnp.int32),
            pltpu.VMEM((TW,), jnp.int32),
            pltpu.VMEM((TW, D_IN), jnp.float32),
            pltpu.SemaphoreType.DMA,
        ],
    )(x, p0, p1)


def _gmm_kernel(tile_r, grp_r, valid_r, offs_r,
                gx_ref, ew_ref, eb_ref, y_ref):
    i = pl.program_id(0)
    t = tile_r[i]
    g = grp_r[i]
    first = jnp.logical_or(i == 0, t != tile_r[jnp.maximum(i - 1, 0)])

    @pl.when(first)
    def _zero():
        y_ref[...] = jnp.zeros_like(y_ref)

    seg0 = offs_r[g]
    seg1 = offs_r[g + 1]
    rows = t * TILE_M + jax.lax.broadcasted_iota(jnp.int32, (TILE_M, 1), 0)
    msk = ((rows >= seg0) & (rows < seg1)).astype(jnp.float32)

    @pl.when(valid_r[i] == 1)
    def _expert():
        sub = jax.lax.broadcasted_iota(jnp.int32, (NUM_EXPERTS, 1), 0)
        eb = jnp.sum(jnp.where(sub == g, eb_ref[...], 0.0), axis=0,
                     keepdims=True)
        y = jnp.dot(gx_ref[...], ew_ref[0],
                    preferred_element_type=jnp.float32) + eb
        y_ref[...] += msk * y


def _moe_matmul(tile_id, grp_id, valid, offs, gx, expert_w, expert_b):
    grid_spec = pltpu.PrefetchScalarGridSpec(
        num_scalar_prefetch=4,
        grid=(MAX_STEPS,),
        in_specs=[
            pl.BlockSpec((TILE_M, D_IN),
                         lambda i, tr, gr, vr, orf: (tr[i], 0)),
            pl.BlockSpec((1, D_IN, D_OUT),
                         lambda i, tr, gr, vr, orf: (gr[i], 0, 0)),
            pl.BlockSpec((NUM_EXPERTS, D_OUT),
                         lambda i, tr, gr, vr, orf: (0, 0)),
        ],
        out_specs=pl.BlockSpec((TILE_M, D_OUT),
                               lambda i, tr, gr, vr, orf: (tr[i], 0)),
    )
    return pl.pallas_call(
        _gmm_kernel,
        grid_spec=grid_spec,
        out_shape=jax.ShapeDtypeStruct((A, D_OUT), jnp.float32),
        compiler_params=pltpu.CompilerParams(
            dimension_semantics=("arbitrary",)),
    )(tile_id, grp_id, valid, offs, gx, expert_w, expert_b)


def _res_kernel(x_ref, rw_ref, rb_ref, cr_ref, yres_ref):
    y = jnp.dot(x_ref[...], rw_ref[...],
                preferred_element_type=jnp.float32) + rb_ref[...][None, :]
    yres_ref[...] = cr_ref[...] * y


def _res_matmul(x, res_w, res_b, cr):
    m_tiles = T // TILE
    return pl.pallas_call(
        _res_kernel,
        grid=(m_tiles,),
        in_specs=[
            pl.BlockSpec((TILE, D_IN), lambda m: (m, 0)),
            pl.BlockSpec((D_IN, D_OUT), lambda m: (0, 0)),
            pl.BlockSpec((D_OUT,), lambda m: (0,)),
            pl.BlockSpec((TILE, 1), lambda m: (m, 0)),
        ],
        out_specs=pl.BlockSpec((TILE, D_OUT), lambda m: (m, 0)),
        out_shape=jax.ShapeDtypeStruct((T, D_OUT), jnp.float32),
    )(x, res_w, res_b, cr)


def _gather_body(y_hbm, p0_hbm, p1_hbm, y1_hbm, y2_hbm,
                 idx0_v, idx1_v, rows_v, sem):
    wid = lax.axis_index("s") * NC + lax.axis_index("c")
    base = wid * TW
    pltpu.sync_copy(p0_hbm.at[pl.ds(base, TW)], idx0_v)
    pltpu.sync_copy(p1_hbm.at[pl.ds(base, TW)], idx1_v)
    pltpu.async_copy(y_hbm.at[idx0_v], rows_v, sem).wait()
    pltpu.sync_copy(rows_v, y1_hbm.at[pl.ds(base, TW)])
    pltpu.async_copy(y_hbm.at[idx1_v], rows_v, sem).wait()
    pltpu.sync_copy(rows_v, y2_hbm.at[pl.ds(base, TW)])


def _sc_gather(y, p0, p1):
    mesh = plsc.VectorSubcoreMesh(core_axis_name="c", subcore_axis_name="s")
    return pl.kernel(
        _gather_body,
        out_type=[
            jax.ShapeDtypeStruct((T, D_OUT), jnp.float32),
            jax.ShapeDtypeStruct((T, D_OUT), jnp.float32),
        ],
        mesh=mesh,
        scratch_types=[
            pltpu.VMEM((TW,), jnp.int32),
            pltpu.VMEM((TW,), jnp.int32),
            pltpu.VMEM((TW, D_OUT), jnp.float32),
            pltpu.SemaphoreType.DMA,
        ],
    )(y, p0, p1)


def _combine_kernel(y1_ref, y2_ref, yres_ref, c0_ref, c1_ref, out_ref):
    out_ref[...] = (c0_ref[...] * y1_ref[...] + c1_ref[...] * y2_ref[...]
                    + yres_ref[...])


def _combine(y1, y2, yres, c0, c1):
    m_tiles = T // TILE
    return pl.pallas_call(
        _combine_kernel,
        grid=(m_tiles,),
        in_specs=[
            pl.BlockSpec((TILE, D_OUT), lambda m: (m, 0)),
            pl.BlockSpec((TILE, D_OUT), lambda m: (m, 0)),
            pl.BlockSpec((TILE, D_OUT), lambda m: (m, 0)),
            pl.BlockSpec((TILE, 1), lambda m: (m, 0)),
            pl.BlockSpec((TILE, 1), lambda m: (m, 0)),
        ],
        out_specs=pl.BlockSpec((TILE, D_OUT), lambda m: (m, 0)),
        out_shape=jax.ShapeDtypeStruct((T, D_OUT), jnp.float32),
    )(y1, y2, yres, c0, c1)


def _step_list(offs_v):
    t_ar = jnp.arange(N_TILES, dtype=jnp.int32)
    g_ar = jnp.arange(NUM_EXPERTS, dtype=jnp.int32)
    ts = t_ar[:, None] * TILE_M
    s0 = offs_v[None, :NUM_EXPERTS]
    s1 = offs_v[None, 1:NUM_EXPERTS + 1]
    big = jnp.int32(2 ** 30)
    m = (s0 < ts + TILE_M) & (s1 > ts)
    keys = jnp.where(m, t_ar[:, None] * 16 + g_ar[None, :], big).reshape(-1)
    keys = jnp.sort(keys)[:MAX_STEPS]
    valid = (keys < big).astype(jnp.int32)
    tile_id = jnp.where(valid == 1, keys >> 4, N_TILES - 1).astype(jnp.int32)
    grp_id = jnp.where(valid == 1, keys & 15, NUM_EXPERTS - 1).astype(jnp.int32)
    return tile_id, grp_id, valid


def kernel(x, gate_w, expert_w, expert_b, res_gate_w, res_w, res_b, weight_w):
    del res_gate_w
    p0_2d, p1_2d, c0_2d, c1_2d, cr_2d, counts16 = _gating(x, gate_w, weight_w)
    p0 = p0_2d.reshape(T)
    p1 = p1_2d.reshape(T)
    counts = counts16[0, :NUM_EXPERTS]
    offs_v = jnp.concatenate([
        jnp.zeros((1,), jnp.int32),
        jnp.cumsum(counts),
    ])
    tile_id, grp_id, valid = _step_list(offs_v)
    gx = _sc_scatter(x, p0, p1)
    yres = _res_matmul(x, res_w, res_b, cr_2d)
    y = _moe_matmul(tile_id, grp_id, valid, offs_v, gx, expert_w, expert_b)
    y1, y2 = _sc_gather(y, p0, p1)
    return _combine(y1, y2, yres, c0_2d, c1_2d)


# speedup vs baseline: 1.1041x; 1.0279x over previous
import functools

import jax
import jax.numpy as jnp
from jax import lax
from jax.experimental import pallas as pl
from jax.experimental.pallas import tpu as pltpu
from jax.experimental.pallas import tpu_sc as plsc

D_IN = 1024
D_OUT = 1024
NUM_EXPERTS = 8
T = 2048
A = 2 * T
R = A + T

TILE = 256
TILE_M = 256
N_TILES = A // TILE_M
MAX_STEPS = N_TILES + NUM_EXPERTS - 1

NC = 2
NS = 16
NW = NC * NS
TW = T // NW


def _gating_kernel(x_ref, gate_w_ref, weight_w_ref,
                   p0_ref, p1_ref, c0_ref, c1_ref, cr_ref, offs_ref):
    x = x_ref[...]
    logits = jnp.dot(x, gate_w_ref[...], preferred_element_type=jnp.float32)
    probs = jax.nn.softmax(logits, axis=-1)
    lane = jax.lax.broadcasted_iota(jnp.int32, probs.shape, 1)
    i1 = jnp.argmax(probs, axis=-1)[:, None]
    v1 = jnp.max(probs, axis=-1, keepdims=True)
    probs2 = jnp.where(lane == i1, -1.0, probs)
    i2 = jnp.argmax(probs2, axis=-1)[:, None]
    v2 = jnp.max(probs2, axis=-1, keepdims=True)
    oh1 = (lane == i1).astype(jnp.float32)
    oh2 = (lane == i2).astype(jnp.float32)

    cnt = oh1 + oh2
    row = jax.lax.broadcasted_iota(jnp.int32, (T, NUM_EXPERTS), 0)
    cs = cnt
    sh = 1
    while sh < T:
        cs = cs + jnp.where(row >= sh, pltpu.roll(cs, sh, 0), 0.0)
        sh *= 2
    excl = cs - cnt
    totals = cs[T - 1:T, :]

    base1 = jnp.sum(jnp.where(lane < i1, totals, 0.0), axis=-1, keepdims=True)
    base2 = jnp.sum(jnp.where(lane < i2, totals, 0.0), axis=-1, keepdims=True)
    rank1 = jnp.sum(oh1 * excl, axis=-1, keepdims=True)
    rank2 = jnp.sum(oh2 * excl, axis=-1, keepdims=True)
    p0 = base1 + rank1
    p1 = base2 + rank2
    p0_ref[...] = p0.astype(jnp.int32)
    p1_ref[...] = p1.astype(jnp.int32)

    wlog = jnp.dot(x, weight_w_ref[...], preferred_element_type=jnp.float32)
    ow = jax.nn.softmax(wlog, axis=-1)
    ow0 = ow[:, 0:1]
    c0_ref[...] = ow0 * v1
    c1_ref[...] = ow0 * v2
    cr_ref[...] = ow[:, 1:2]

    offs_ref[...] = jnp.concatenate([totals, totals * 0.0],
                                    axis=1).astype(jnp.int32)


def _gating(x, gate_w, weight_w):
    return pl.pallas_call(
        _gating_kernel,
        grid=(1,),
        in_specs=[
            pl.BlockSpec((T, D_IN), lambda i: (0, 0)),
            pl.BlockSpec((D_IN, NUM_EXPERTS), lambda i: (0, 0)),
            pl.BlockSpec((D_IN, 2), lambda i: (0, 0)),
        ],
        out_specs=[
            pl.BlockSpec((T, 1), lambda i: (0, 0)),
            pl.BlockSpec((T, 1), lambda i: (0, 0)),
            pl.BlockSpec((T, 1), lambda i: (0, 0)),
            pl.BlockSpec((T, 1), lambda i: (0, 0)),
            pl.BlockSpec((T, 1), lambda i: (0, 0)),
            pl.BlockSpec((1, 16), lambda i: (0, 0)),
        ],
        out_shape=[
            jax.ShapeDtypeStruct((T, 1), jnp.int32),
            jax.ShapeDtypeStruct((T, 1), jnp.int32),
            jax.ShapeDtypeStruct((T, 1), jnp.float32),
            jax.ShapeDtypeStruct((T, 1), jnp.float32),
            jax.ShapeDtypeStruct((T, 1), jnp.float32),
            jax.ShapeDtypeStruct((1, 16), jnp.int32),
        ],
    )(x, gate_w, weight_w)


def _scatter_body(x_hbm, p0_hbm, p1_hbm, gx_hbm, idx0_v, idx1_v, rows_v, sem):
    wid = lax.axis_index("s") * NC + lax.axis_index("c")
    base = wid * TW
    pltpu.sync_copy(p0_hbm.at[pl.ds(base, TW)], idx0_v)
    pltpu.sync_copy(p1_hbm.at[pl.ds(base, TW)], idx1_v)
    pltpu.sync_copy(x_hbm.at[pl.ds(base, TW)], rows_v)
    pltpu.async_copy(rows_v, gx_hbm.at[idx0_v], sem).wait()
    pltpu.async_copy(rows_v, gx_hbm.at[idx1_v], sem).wait()


def _sc_scatter(x, p0, p1):
    mesh = plsc.VectorSubcoreMesh(core_axis_name="c", subcore_axis_name="s")
    return pl.kernel(
        _scatter_body,
        out_type=jax.ShapeDtypeStruct((A, D_IN), jnp.float32),
        mesh=mesh,
        scratch_types=[
            pltpu.VMEM((TW,), jnp.int32),
            pltpu.VMEM((TW,), jnp.int32),
            pltpu.VMEM((TW, D_IN), jnp.float32),
            pltpu.SemaphoreType.DMA,
        ],
    )(x, p0, p1)


def _gmm_kernel(tile_r, grp_r, valid_r, offs_r,
                gx_ref, ew_ref, eb_ref, y_ref):
    i = pl.program_id(0)
    t = tile_r[i]
    g = grp_r[i]
    first = jnp.logical_or(i == 0, t != tile_r[jnp.maximum(i - 1, 0)])

    @pl.when(first)
    def _zero():
        y_ref[...] = jnp.zeros_like(y_ref)

    seg0 = offs_r[g]
    seg1 = offs_r[g + 1]
    rows = t * TILE_M + jax.lax.broadcasted_iota(jnp.int32, (TILE_M, 1), 0)
    msk = ((rows >= seg0) & (rows < seg1)).astype(jnp.float32)

    @pl.when(valid_r[i] == 1)
    def _expert():
        sub = jax.lax.broadcasted_iota(jnp.int32, (NUM_EXPERTS, 1), 0)
        eb = jnp.sum(jnp.where(sub == g, eb_ref[...], 0.0), axis=0,
                     keepdims=True)
        y = jnp.dot(gx_ref[...], ew_ref[0],
                    preferred_element_type=jnp.float32) + eb
        y_ref[...] += msk * y


def _moe_matmul(tile_id, grp_id, valid, offs, gx, expert_w, expert_b):
    grid_spec = pltpu.PrefetchScalarGridSpec(
        num_scalar_prefetch=4,
        grid=(MAX_STEPS,),
        in_specs=[
            pl.BlockSpec((TILE_M, D_IN),
                         lambda i, tr, gr, vr, orf: (tr[i], 0)),
            pl.BlockSpec((1, D_IN, D_OUT),
                         lambda i, tr, gr, vr, orf: (gr[i], 0, 0)),
            pl.BlockSpec((NUM_EXPERTS, D_OUT),
                         lambda i, tr, gr, vr, orf: (0, 0)),
        ],
        out_specs=pl.BlockSpec((TILE_M, D_OUT),
                               lambda i, tr, gr, vr, orf: (tr[i], 0)),
    )
    return pl.pallas_call(
        _gmm_kernel,
        grid_spec=grid_spec,
        out_shape=jax.ShapeDtypeStruct((A, D_OUT), jnp.float32),
        compiler_params=pltpu.CompilerParams(
            dimension_semantics=("arbitrary",)),
    )(tile_id, grp_id, valid, offs, gx, expert_w, expert_b)


def _res_kernel(x_ref, rw_ref, rb_ref, cr_ref, yres_ref):
    y = jnp.dot(x_ref[...], rw_ref[...],
                preferred_element_type=jnp.float32) + rb_ref[...][None, :]
    yres_ref[...] = cr_ref[...] * y


def _res_matmul(x, res_w, res_b, cr):
    m_tiles = T // TILE
    return pl.pallas_call(
        _res_kernel,
        grid=(m_tiles,),
        in_specs=[
            pl.BlockSpec((TILE, D_IN), lambda m: (m, 0)),
            pl.BlockSpec((D_IN, D_OUT), lambda m: (0, 0)),
            pl.BlockSpec((D_OUT,), lambda m: (0,)),
            pl.BlockSpec((TILE, 1), lambda m: (m, 0)),
        ],
        out_specs=pl.BlockSpec((TILE, D_OUT), lambda m: (m, 0)),
        out_shape=jax.ShapeDtypeStruct((T, D_OUT), jnp.float32),
    )(x, res_w, res_b, cr)


def _gather_body(y_hbm, p0_hbm, p1_hbm, y1_hbm, y2_hbm,
                 idx0_v, idx1_v, rows_v, sem):
    wid = lax.axis_index("s") * NC + lax.axis_index("c")
    base = wid * TW
    pltpu.sync_copy(p0_hbm.at[pl.ds(base, TW)], idx0_v)
    pltpu.sync_copy(p1_hbm.at[pl.ds(base, TW)], idx1_v)
    pltpu.async_copy(y_hbm.at[idx0_v], rows_v, sem).wait()
    pltpu.sync_copy(rows_v, y1_hbm.at[pl.ds(base, TW)])
    pltpu.async_copy(y_hbm.at[idx1_v], rows_v, sem).wait()
    pltpu.sync_copy(rows_v, y2_hbm.at[pl.ds(base, TW)])


def _sc_gather(y, p0, p1):
    mesh = plsc.VectorSubcoreMesh(core_axis_name="c", subcore_axis_name="s")
    return pl.kernel(
        _gather_body,
        out_type=[
            jax.ShapeDtypeStruct((T, D_OUT), jnp.float32),
            jax.ShapeDtypeStruct((T, D_OUT), jnp.float32),
        ],
        mesh=mesh,
        scratch_types=[
            pltpu.VMEM((TW,), jnp.int32),
            pltpu.VMEM((TW,), jnp.int32),
            pltpu.VMEM((TW, D_OUT), jnp.float32),
            pltpu.SemaphoreType.DMA,
        ],
    )(y, p0, p1)


def _combine_kernel(y1_ref, y2_ref, yres_ref, c0_ref, c1_ref, out_ref):
    out_ref[...] = (c0_ref[...] * y1_ref[...] + c1_ref[...] * y2_ref[...]
                    + yres_ref[...])


def _combine(y1, y2, yres, c0, c1):
    m_tiles = T // TILE
    return pl.pallas_call(
        _combine_kernel,
        grid=(m_tiles,),
        in_specs=[
            pl.BlockSpec((TILE, D_OUT), lambda m: (m, 0)),
            pl.BlockSpec((TILE, D_OUT), lambda m: (m, 0)),
            pl.BlockSpec((TILE, D_OUT), lambda m: (m, 0)),
            pl.BlockSpec((TILE, 1), lambda m: (m, 0)),
            pl.BlockSpec((TILE, 1), lambda m: (m, 0)),
        ],
        out_specs=pl.BlockSpec((TILE, D_OUT), lambda m: (m, 0)),
        out_shape=jax.ShapeDtypeStruct((T, D_OUT), jnp.float32),
    )(y1, y2, yres, c0, c1)


def _step_list(offs_v):
    t_ar = jnp.arange(N_TILES, dtype=jnp.int32)
    g_ar = jnp.arange(NUM_EXPERTS, dtype=jnp.int32)
    ts = t_ar[:, None] * TILE_M
    s0 = offs_v[None, :NUM_EXPERTS]
    s1 = offs_v[None, 1:NUM_EXPERTS + 1]
    big = jnp.int32(2 ** 30)
    m = (s0 < ts + TILE_M) & (s1 > ts)
    keys = jnp.where(m, t_ar[:, None] * 16 + g_ar[None, :], big).reshape(-1)
    keys = jnp.sort(keys)[:MAX_STEPS]
    valid = (keys < big).astype(jnp.int32)
    tile_id = jnp.where(valid == 1, keys >> 4, N_TILES - 1).astype(jnp.int32)
    grp_id = jnp.where(valid == 1, keys & 15, NUM_EXPERTS - 1).astype(jnp.int32)
    return tile_id, grp_id, valid


def kernel(x, gate_w, expert_w, expert_b, res_gate_w, res_w, res_b, weight_w):
    del res_gate_w
    p0_2d, p1_2d, c0_2d, c1_2d, cr_2d, counts16 = _gating(x, gate_w, weight_w)
    p0 = p0_2d.reshape(T)
    p1 = p1_2d.reshape(T)
    counts = counts16[0, :NUM_EXPERTS]
    offs_v = jnp.concatenate([
        jnp.zeros((1,), jnp.int32),
        jnp.cumsum(counts),
    ])
    tile_id, grp_id, valid = _step_list(offs_v)
    gx = _sc_scatter(x, p0, p1)
    yres = _res_matmul(x, res_w, res_b, cr_2d)
    y = _moe_matmul(tile_id, grp_id, valid, offs_v, gx, expert_w, expert_b)
    y1, y2 = _sc_gather(y, p0, p1)
    return _combine(y1, y2, yres, c0_2d, c1_2d)


# speedup vs baseline: 1.1159x; 1.0107x over previous
import functools

import jax
import jax.numpy as jnp
from jax import lax
from jax.experimental import pallas as pl
from jax.experimental.pallas import tpu as pltpu
from jax.experimental.pallas import tpu_sc as plsc

D_IN = 1024
D_OUT = 1024
NUM_EXPERTS = 8
T = 2048
A = 2 * T
R = A + T

TILE = 256
TILE_M = 256
N_TILES = A // TILE_M + NUM_EXPERTS
GXR = N_TILES * TILE_M

NC = 2
NS = 16
NW = NC * NS
TW = T // NW


def _gating_kernel(x_ref, gate_w_ref, weight_w_ref,
                   p0_ref, p1_ref, c0_ref, c1_ref, cr_ref, offs_ref):
    x = x_ref[...]
    logits = jnp.dot(x, gate_w_ref[...], preferred_element_type=jnp.float32)
    probs = jax.nn.softmax(logits, axis=-1)
    lane = jax.lax.broadcasted_iota(jnp.int32, probs.shape, 1)
    i1 = jnp.argmax(probs, axis=-1)[:, None]
    v1 = jnp.max(probs, axis=-1, keepdims=True)
    probs2 = jnp.where(lane == i1, -1.0, probs)
    i2 = jnp.argmax(probs2, axis=-1)[:, None]
    v2 = jnp.max(probs2, axis=-1, keepdims=True)
    oh1 = (lane == i1).astype(jnp.float32)
    oh2 = (lane == i2).astype(jnp.float32)

    cnt = oh1 + oh2
    row = jax.lax.broadcasted_iota(jnp.int32, (T, NUM_EXPERTS), 0)
    cs = cnt
    sh = 1
    while sh < T:
        cs = cs + jnp.where(row >= sh, pltpu.roll(cs, sh, 0), 0.0)
        sh *= 2
    excl = cs - cnt
    totals = cs[T - 1:T, :]
    tpad = jnp.floor((totals + float(TILE_M - 1)) * (1.0 / TILE_M)) * TILE_M

    base1 = jnp.sum(jnp.where(lane < i1, tpad, 0.0), axis=-1, keepdims=True)
    base2 = jnp.sum(jnp.where(lane < i2, tpad, 0.0), axis=-1, keepdims=True)
    rank1 = jnp.sum(oh1 * excl, axis=-1, keepdims=True)
    rank2 = jnp.sum(oh2 * excl, axis=-1, keepdims=True)
    p0 = base1 + rank1
    p1 = base2 + rank2
    p0_ref[...] = p0.astype(jnp.int32)
    p1_ref[...] = p1.astype(jnp.int32)

    wlog = jnp.dot(x, weight_w_ref[...], preferred_element_type=jnp.float32)
    ow = jax.nn.softmax(wlog, axis=-1)
    ow0 = ow[:, 0:1]
    c0_ref[...] = ow0 * v1
    c1_ref[...] = ow0 * v2
    cr_ref[...] = ow[:, 1:2]

    offs_ref[...] = jnp.concatenate([tpad, tpad * 0.0],
                                    axis=1).astype(jnp.int32)


def _gating(x, gate_w, weight_w):
    return pl.pallas_call(
        _gating_kernel,
        grid=(1,),
        in_specs=[
            pl.BlockSpec((T, D_IN), lambda i: (0, 0)),
            pl.BlockSpec((D_IN, NUM_EXPERTS), lambda i: (0, 0)),
            pl.BlockSpec((D_IN, 2), lambda i: (0, 0)),
        ],
        out_specs=[
            pl.BlockSpec((T, 1), lambda i: (0, 0)),
            pl.BlockSpec((T, 1), lambda i: (0, 0)),
            pl.BlockSpec((T, 1), lambda i: (0, 0)),
            pl.BlockSpec((T, 1), lambda i: (0, 0)),
            pl.BlockSpec((T, 1), lambda i: (0, 0)),
            pl.BlockSpec((1, 16), lambda i: (0, 0)),
        ],
        out_shape=[
            jax.ShapeDtypeStruct((T, 1), jnp.int32),
            jax.ShapeDtypeStruct((T, 1), jnp.int32),
            jax.ShapeDtypeStruct((T, 1), jnp.float32),
            jax.ShapeDtypeStruct((T, 1), jnp.float32),
            jax.ShapeDtypeStruct((T, 1), jnp.float32),
            jax.ShapeDtypeStruct((1, 16), jnp.int32),
        ],
    )(x, gate_w, weight_w)


def _scatter_body(x_hbm, p0_hbm, p1_hbm, gx_hbm, idx0_v, idx1_v, rows_v, sem):
    wid = lax.axis_index("s") * NC + lax.axis_index("c")
    base = wid * TW
    pltpu.sync_copy(p0_hbm.at[pl.ds(base, TW)], idx0_v)
    pltpu.sync_copy(p1_hbm.at[pl.ds(base, TW)], idx1_v)
    pltpu.sync_copy(x_hbm.at[pl.ds(base, TW)], rows_v)
    pltpu.async_copy(rows_v, gx_hbm.at[idx0_v], sem).wait()
    pltpu.async_copy(rows_v, gx_hbm.at[idx1_v], sem).wait()


def _sc_scatter(x, p0, p1):
    mesh = plsc.VectorSubcoreMesh(core_axis_name="c", subcore_axis_name="s")
    return pl.kernel(
        _scatter_body,
        out_type=jax.ShapeDtypeStruct((GXR, D_IN), jnp.float32),
        mesh=mesh,
        scratch_types=[
            pltpu.VMEM((TW,), jnp.int32),
            pltpu.VMEM((TW,), jnp.int32),
            pltpu.VMEM((TW, D_IN), jnp.float32),
            pltpu.SemaphoreType.DMA,
        ],
    )(x, p0, p1)


def _gmm_kernel(grp_r, valid_r, nvt_r, gx_ref, ew_ref, eb_ref, y_ref):
    i = pl.program_id(0)
    g = grp_r[i]

    @pl.when(valid_r[i] == 1)
    def _expert():
        sub = jax.lax.broadcasted_iota(jnp.int32, (NUM_EXPERTS, 1), 0)
        eb = jnp.sum(jnp.where(sub == g, eb_ref[...], 0.0), axis=0,
                     keepdims=True)
        y_ref[...] = jnp.dot(gx_ref[...], ew_ref[0],
                             preferred_element_type=jnp.float32) + eb


def _moe_matmul(grp_id, valid, nvt, gx, expert_w, expert_b):
    grid_spec = pltpu.PrefetchScalarGridSpec(
        num_scalar_prefetch=3,
        grid=(N_TILES,),
        in_specs=[
            pl.BlockSpec((TILE_M, D_IN),
                         lambda i, gr, vr, nv: (jnp.minimum(i, nv[0] - 1), 0)),
            pl.BlockSpec((1, D_IN, D_OUT),
                         lambda i, gr, vr, nv: (gr[i], 0, 0)),
            pl.BlockSpec((NUM_EXPERTS, D_OUT),
                         lambda i, gr, vr, nv: (0, 0)),
        ],
        out_specs=pl.BlockSpec((TILE_M, D_OUT),
                               lambda i, gr, vr, nv: (jnp.minimum(i, nv[0] - 1), 0)),
    )
    return pl.pallas_call(
        _gmm_kernel,
        grid_spec=grid_spec,
        out_shape=jax.ShapeDtypeStruct((GXR, D_OUT), jnp.float32),
        compiler_params=pltpu.CompilerParams(
            dimension_semantics=("arbitrary",)),
    )(grp_id, valid, nvt, gx, expert_w, expert_b)


def _res_kernel(x_ref, rw_ref, rb_ref, cr_ref, yres_ref):
    y = jnp.dot(x_ref[...], rw_ref[...],
                preferred_element_type=jnp.float32) + rb_ref[...][None, :]
    yres_ref[...] = cr_ref[...] * y


def _res_matmul(x, res_w, res_b, cr):
    m_tiles = T // TILE
    return pl.pallas_call(
        _res_kernel,
        grid=(m_tiles,),
        in_specs=[
            pl.BlockSpec((TILE, D_IN), lambda m: (m, 0)),
            pl.BlockSpec((D_IN, D_OUT), lambda m: (0, 0)),
            pl.BlockSpec((D_OUT,), lambda m: (0,)),
            pl.BlockSpec((TILE, 1), lambda m: (m, 0)),
        ],
        out_specs=pl.BlockSpec((TILE, D_OUT), lambda m: (m, 0)),
        out_shape=jax.ShapeDtypeStruct((T, D_OUT), jnp.float32),
    )(x, res_w, res_b, cr)


def _gather_body(y_hbm, p0_hbm, p1_hbm, y1_hbm, y2_hbm,
                 idx0_v, idx1_v, rows_v, sem):
    wid = lax.axis_index("s") * NC + lax.axis_index("c")
    base = wid * TW
    pltpu.sync_copy(p0_hbm.at[pl.ds(base, TW)], idx0_v)
    pltpu.sync_copy(p1_hbm.at[pl.ds(base, TW)], idx1_v)
    pltpu.async_copy(y_hbm.at[idx0_v], rows_v, sem).wait()
    pltpu.sync_copy(rows_v, y1_hbm.at[pl.ds(base, TW)])
    pltpu.async_copy(y_hbm.at[idx1_v], rows_v, sem).wait()
    pltpu.sync_copy(rows_v, y2_hbm.at[pl.ds(base, TW)])


def _sc_gather(y, p0, p1):
    mesh = plsc.VectorSubcoreMesh(core_axis_name="c", subcore_axis_name="s")
    return pl.kernel(
        _gather_body,
        out_type=[
            jax.ShapeDtypeStruct((T, D_OUT), jnp.float32),
            jax.ShapeDtypeStruct((T, D_OUT), jnp.float32),
        ],
        mesh=mesh,
        scratch_types=[
            pltpu.VMEM((TW,), jnp.int32),
            pltpu.VMEM((TW,), jnp.int32),
            pltpu.VMEM((TW, D_OUT), jnp.float32),
            pltpu.SemaphoreType.DMA,
        ],
    )(y, p0, p1)


def _combine_kernel(y1_ref, y2_ref, yres_ref, c0_ref, c1_ref, out_ref):
    out_ref[...] = (c0_ref[...] * y1_ref[...] + c1_ref[...] * y2_ref[...]
                    + yres_ref[...])


def _combine(y1, y2, yres, c0, c1):
    m_tiles = T // TILE
    return pl.pallas_call(
        _combine_kernel,
        grid=(m_tiles,),
        in_specs=[
            pl.BlockSpec((TILE, D_OUT), lambda m: (m, 0)),
            pl.BlockSpec((TILE, D_OUT), lambda m: (m, 0)),
            pl.BlockSpec((TILE, D_OUT), lambda m: (m, 0)),
            pl.BlockSpec((TILE, 1), lambda m: (m, 0)),
            pl.BlockSpec((TILE, 1), lambda m: (m, 0)),
        ],
        out_specs=pl.BlockSpec((TILE, D_OUT), lambda m: (m, 0)),
        out_shape=jax.ShapeDtypeStruct((T, D_OUT), jnp.float32),
    )(y1, y2, yres, c0, c1)


def _tile_groups(tpad_counts):
    offs_pad = jnp.concatenate([jnp.zeros((1,), jnp.int32),
                                jnp.cumsum(tpad_counts)])
    ts = jnp.arange(N_TILES, dtype=jnp.int32) * TILE_M
    grp = jnp.sum((offs_pad[None, 1:NUM_EXPERTS + 1] <= ts[:, None])
                  .astype(jnp.int32), axis=1)
    grp = jnp.clip(grp, 0, NUM_EXPERTS - 1).astype(jnp.int32)
    valid = (ts < offs_pad[NUM_EXPERTS]).astype(jnp.int32)
    nvt = jnp.maximum(offs_pad[NUM_EXPERTS] // TILE_M, 1)[None].astype(jnp.int32)
    return grp, valid, nvt


def kernel(x, gate_w, expert_w, expert_b, res_gate_w, res_w, res_b, weight_w):
    del res_gate_w
    p0_2d, p1_2d, c0_2d, c1_2d, cr_2d, counts16 = _gating(x, gate_w, weight_w)
    p0 = p0_2d.reshape(T)
    p1 = p1_2d.reshape(T)
    counts = counts16[0, :NUM_EXPERTS]
    grp_id, valid, nvt = _tile_groups(counts)
    gx = _sc_scatter(x, p0, p1)
    yres = _res_matmul(x, res_w, res_b, cr_2d)
    y = _moe_matmul(grp_id, valid, nvt, gx, expert_w, expert_b)
    y1, y2 = _sc_gather(y, p0, p1)
    return _combine(y1, y2, yres, c0_2d, c1_2d)


# speedup vs baseline: 1.1253x; 1.0085x over previous
import functools

import jax
import jax.numpy as jnp
from jax import lax
from jax.experimental import pallas as pl
from jax.experimental.pallas import tpu as pltpu
from jax.experimental.pallas import tpu_sc as plsc

D_IN = 1024
D_OUT = 1024
NUM_EXPERTS = 8
T = 2048
A = 2 * T
R = A + T

TILE = 256
TILE_M = 256
N_TILES = A // TILE_M + NUM_EXPERTS
GXR = N_TILES * TILE_M

NC = 2
NS = 16
NW = NC * NS
TW = T // NW


def _gating_kernel(x_ref, gate_w_ref, weight_w_ref,
                   p0_ref, p1_ref, c0_ref, c1_ref, cr_ref, offs_ref):
    x = x_ref[...]
    logits = jnp.dot(x, gate_w_ref[...], preferred_element_type=jnp.float32)
    probs = jax.nn.softmax(logits, axis=-1)
    lane = jax.lax.broadcasted_iota(jnp.int32, probs.shape, 1)
    i1 = jnp.argmax(probs, axis=-1)[:, None]
    v1 = jnp.max(probs, axis=-1, keepdims=True)
    probs2 = jnp.where(lane == i1, -1.0, probs)
    i2 = jnp.argmax(probs2, axis=-1)[:, None]
    v2 = jnp.max(probs2, axis=-1, keepdims=True)
    oh1 = (lane == i1).astype(jnp.float32)
    oh2 = (lane == i2).astype(jnp.float32)

    cnt = oh1 + oh2
    row = jax.lax.broadcasted_iota(jnp.int32, (T, NUM_EXPERTS), 0)
    cs = cnt
    sh = 1
    while sh < T:
        cs = cs + jnp.where(row >= sh, pltpu.roll(cs, sh, 0), 0.0)
        sh *= 2
    excl = cs - cnt
    totals = cs[T - 1:T, :]
    tpad = jnp.floor((totals + float(TILE_M - 1)) * (1.0 / TILE_M)) * TILE_M

    base1 = jnp.sum(jnp.where(lane < i1, tpad, 0.0), axis=-1, keepdims=True)
    base2 = jnp.sum(jnp.where(lane < i2, tpad, 0.0), axis=-1, keepdims=True)
    rank1 = jnp.sum(oh1 * excl, axis=-1, keepdims=True)
    rank2 = jnp.sum(oh2 * excl, axis=-1, keepdims=True)
    p0 = base1 + rank1
    p1 = base2 + rank2
    p0_ref[...] = p0.astype(jnp.int32)
    p1_ref[...] = p1.astype(jnp.int32)

    wlog = jnp.dot(x, weight_w_ref[...], preferred_element_type=jnp.float32)
    ow = jax.nn.softmax(wlog, axis=-1)
    ow0 = ow[:, 0:1]
    c0_ref[...] = ow0 * v1
    c1_ref[...] = ow0 * v2
    cr_ref[...] = ow[:, 1:2]

    offs_ref[...] = jnp.concatenate([tpad, tpad * 0.0],
                                    axis=1).astype(jnp.int32)


def _gating(x, gate_w, weight_w):
    return pl.pallas_call(
        _gating_kernel,
        grid=(1,),
        in_specs=[
            pl.BlockSpec((T, D_IN), lambda i: (0, 0)),
            pl.BlockSpec((D_IN, NUM_EXPERTS), lambda i: (0, 0)),
            pl.BlockSpec((D_IN, 2), lambda i: (0, 0)),
        ],
        out_specs=[
            pl.BlockSpec((T, 1), lambda i: (0, 0)),
            pl.BlockSpec((T, 1), lambda i: (0, 0)),
            pl.BlockSpec((T, 1), lambda i: (0, 0)),
            pl.BlockSpec((T, 1), lambda i: (0, 0)),
            pl.BlockSpec((T, 1), lambda i: (0, 0)),
            pl.BlockSpec((1, 16), lambda i: (0, 0)),
        ],
        out_shape=[
            jax.ShapeDtypeStruct((T, 1), jnp.int32),
            jax.ShapeDtypeStruct((T, 1), jnp.int32),
            jax.ShapeDtypeStruct((T, 1), jnp.float32),
            jax.ShapeDtypeStruct((T, 1), jnp.float32),
            jax.ShapeDtypeStruct((T, 1), jnp.float32),
            jax.ShapeDtypeStruct((1, 16), jnp.int32),
        ],
    )(x, gate_w, weight_w)


def _scatter_body(x_hbm, p0_hbm, p1_hbm, gx_hbm, idx0_v, idx1_v, rows_v, sem):
    wid = lax.axis_index("s") * NC + lax.axis_index("c")
    base = wid * TW
    pltpu.sync_copy(p0_hbm.at[pl.ds(base, TW)], idx0_v)
    pltpu.sync_copy(p1_hbm.at[pl.ds(base, TW)], idx1_v)
    pltpu.sync_copy(x_hbm.at[pl.ds(base, TW)], rows_v)
    pltpu.async_copy(rows_v, gx_hbm.at[idx0_v], sem).wait()
    pltpu.async_copy(rows_v, gx_hbm.at[idx1_v], sem).wait()


def _sc_scatter(x, p0, p1):
    mesh = plsc.VectorSubcoreMesh(core_axis_name="c", subcore_axis_name="s")
    return pl.kernel(
        _scatter_body,
        out_type=jax.ShapeDtypeStruct((GXR, D_IN), jnp.float32),
        mesh=mesh,
        scratch_types=[
            pltpu.VMEM((TW,), jnp.int32),
            pltpu.VMEM((TW,), jnp.int32),
            pltpu.VMEM((TW, D_IN), jnp.float32),
            pltpu.SemaphoreType.DMA,
        ],
    )(x, p0, p1)


def _gmm_kernel(grp_r, valid_r, nvt_r, gx_ref, ew_ref, eb_ref, y_ref):
    i = pl.program_id(0)
    g = grp_r[i]

    @pl.when(valid_r[i] == 1)
    def _expert():
        sub = jax.lax.broadcasted_iota(jnp.int32, (NUM_EXPERTS, 1), 0)
        eb = jnp.sum(jnp.where(sub == g, eb_ref[...], 0.0), axis=0,
                     keepdims=True)
        y_ref[...] = jnp.dot(gx_ref[...], ew_ref[0],
                             preferred_element_type=jnp.float32) + eb


def _moe_matmul(grp_id, valid, nvt, gx, expert_w, expert_b):
    grid_spec = pltpu.PrefetchScalarGridSpec(
        num_scalar_prefetch=3,
        grid=(N_TILES,),
        in_specs=[
            pl.BlockSpec((TILE_M, D_IN),
                         lambda i, gr, vr, nv: (jnp.minimum(i, nv[0] - 1), 0)),
            pl.BlockSpec((1, D_IN, D_OUT),
                         lambda i, gr, vr, nv: (gr[i], 0, 0)),
            pl.BlockSpec((NUM_EXPERTS, D_OUT),
                         lambda i, gr, vr, nv: (0, 0)),
        ],
        out_specs=pl.BlockSpec((TILE_M, D_OUT),
                               lambda i, gr, vr, nv: (jnp.minimum(i, nv[0] - 1), 0)),
    )
    return pl.pallas_call(
        _gmm_kernel,
        grid_spec=grid_spec,
        out_shape=jax.ShapeDtypeStruct((GXR, D_OUT), jnp.float32),
        compiler_params=pltpu.CompilerParams(
            dimension_semantics=("arbitrary",)),
    )(grp_id, valid, nvt, gx, expert_w, expert_b)


def _res_kernel(x_ref, rw_ref, rb_ref, cr_ref, yres_ref):
    y = jnp.dot(x_ref[...], rw_ref[...],
                preferred_element_type=jnp.float32) + rb_ref[...][None, :]
    yres_ref[...] = cr_ref[...] * y


def _res_matmul(x, res_w, res_b, cr):
    m_tiles = T // TILE
    return pl.pallas_call(
        _res_kernel,
        grid=(m_tiles,),
        in_specs=[
            pl.BlockSpec((TILE, D_IN), lambda m: (m, 0)),
            pl.BlockSpec((D_IN, D_OUT), lambda m: (0, 0)),
            pl.BlockSpec((D_OUT,), lambda m: (0,)),
            pl.BlockSpec((TILE, 1), lambda m: (m, 0)),
        ],
        out_specs=pl.BlockSpec((TILE, D_OUT), lambda m: (m, 0)),
        out_shape=jax.ShapeDtypeStruct((T, D_OUT), jnp.float32),
    )(x, res_w, res_b, cr)


def _gather_body(y_hbm, p0_hbm, p1_hbm, y1_hbm, y2_hbm,
                 idx0_v, idx1_v, rows_v, sem):
    wid = lax.axis_index("s") * NC + lax.axis_index("c")
    base = wid * TW
    pltpu.sync_copy(p0_hbm.at[pl.ds(base, TW)], idx0_v)
    pltpu.sync_copy(p1_hbm.at[pl.ds(base, TW)], idx1_v)
    pltpu.async_copy(y_hbm.at[idx0_v], rows_v, sem).wait()
    pltpu.sync_copy(rows_v, y1_hbm.at[pl.ds(base, TW)])
    pltpu.async_copy(y_hbm.at[idx1_v], rows_v, sem).wait()
    pltpu.sync_copy(rows_v, y2_hbm.at[pl.ds(base, TW)])


def _sc_gather(y, p0, p1):
    mesh = plsc.VectorSubcoreMesh(core_axis_name="c", subcore_axis_name="s")
    return pl.kernel(
        _gather_body,
        out_type=[
            jax.ShapeDtypeStruct((T, D_OUT), jnp.float32),
            jax.ShapeDtypeStruct((T, D_OUT), jnp.float32),
        ],
        mesh=mesh,
        scratch_types=[
            pltpu.VMEM((TW,), jnp.int32),
            pltpu.VMEM((TW,), jnp.int32),
            pltpu.VMEM((TW, D_OUT), jnp.float32),
            pltpu.SemaphoreType.DMA,
        ],
    )(y, p0, p1)


def _combine_kernel(y1_ref, y2_ref, yres_ref, c0_ref, c1_ref, out_ref):
    out_ref[...] = (c0_ref[...] * y1_ref[...] + c1_ref[...] * y2_ref[...]
                    + yres_ref[...])


TILE_C = 512


def _combine(y1, y2, yres, c0, c1):
    m_tiles = T // TILE_C
    return pl.pallas_call(
        _combine_kernel,
        grid=(m_tiles,),
        in_specs=[
            pl.BlockSpec((TILE_C, D_OUT), lambda m: (m, 0)),
            pl.BlockSpec((TILE_C, D_OUT), lambda m: (m, 0)),
            pl.BlockSpec((TILE_C, D_OUT), lambda m: (m, 0)),
            pl.BlockSpec((TILE_C, 1), lambda m: (m, 0)),
            pl.BlockSpec((TILE_C, 1), lambda m: (m, 0)),
        ],
        out_specs=pl.BlockSpec((TILE_C, D_OUT), lambda m: (m, 0)),
        out_shape=jax.ShapeDtypeStruct((T, D_OUT), jnp.float32),
    )(y1, y2, yres, c0, c1)


def _tile_groups(tpad_counts):
    offs_pad = jnp.concatenate([jnp.zeros((1,), jnp.int32),
                                jnp.cumsum(tpad_counts)])
    ts = jnp.arange(N_TILES, dtype=jnp.int32) * TILE_M
    grp = jnp.sum((offs_pad[None, 1:NUM_EXPERTS + 1] <= ts[:, None])
                  .astype(jnp.int32), axis=1)
    grp = jnp.clip(grp, 0, NUM_EXPERTS - 1).astype(jnp.int32)
    valid = (ts < offs_pad[NUM_EXPERTS]).astype(jnp.int32)
    nvt = jnp.maximum(offs_pad[NUM_EXPERTS] // TILE_M, 1)[None].astype(jnp.int32)
    return grp, valid, nvt


def kernel(x, gate_w, expert_w, expert_b, res_gate_w, res_w, res_b, weight_w):
    del res_gate_w
    p0_2d, p1_2d, c0_2d, c1_2d, cr_2d, counts16 = _gating(x, gate_w, weight_w)
    p0 = p0_2d.reshape(T)
    p1 = p1_2d.reshape(T)
    counts = counts16[0, :NUM_EXPERTS]
    grp_id, valid, nvt = _tile_groups(counts)
    gx = _sc_scatter(x, p0, p1)
    yres = _res_matmul(x, res_w, res_b, cr_2d)
    y = _moe_matmul(grp_id, valid, nvt, gx, expert_w, expert_b)
    y1, y2 = _sc_gather(y, p0, p1)
    return _combine(y1, y2, yres, c0_2d, c1_2d)
